# Initial kernel scaffold; baseline (speedup 1.0000x reference)
import jax
import jax.numpy as jnp
from jax.experimental import pallas as pl


def kernel(x):
    raise NotImplementedError("write your pallas kernel here")


# speedup vs baseline: 3.6675x; 3.6675x over previous
import functools

import jax
import jax.numpy as jnp
from jax.experimental import pallas as pl
from jax.experimental.pallas import tpu as pltpu

ROWS = 128
COLS = 32768
BLK = 4096
GRID = COLS // BLK
INT_MAX = 2**31 - 1


def _argmax_body(x_ref, out_ref, rmax_ref, ridx_ref):
    b = pl.program_id(0)

    @pl.when(b == 0)
    def _init():
        rmax_ref[0] = -jnp.inf
        ridx_ref[0] = jnp.int32(INT_MAX)

    xb = x_ref[...]
    m = jnp.max(xb)

    @pl.when(m >= rmax_ref[0])
    def _update():
        rows = jax.lax.broadcasted_iota(jnp.int32, (ROWS, BLK), 0)
        cols = jax.lax.broadcasted_iota(jnp.int32, (ROWS, BLK), 1)
        flat = rows * COLS + (b * BLK + cols)
        cand = jnp.min(jnp.where(xb == m, flat, jnp.int32(INT_MAX)))
        old_m = rmax_ref[0]
        old_i = ridx_ref[0]
        better = (m > old_m) | (cand < old_i)
        ridx_ref[0] = jnp.where(better, cand, old_i)
        rmax_ref[0] = jnp.where(m > old_m, m, old_m)

    @pl.when(b == GRID - 1)
    def _fin():
        out_ref[0] = ridx_ref[0]


def kernel(x):
    out = pl.pallas_call(
        _argmax_body,
        grid=(GRID,),
        in_specs=[pl.BlockSpec((ROWS, BLK), lambda b: (0, b))],
        out_specs=pl.BlockSpec(memory_space=pltpu.SMEM),
        out_shape=jax.ShapeDtypeStruct((1,), jnp.int32),
        scratch_shapes=[
            pltpu.SMEM((1,), jnp.float32),
            pltpu.SMEM((1,), jnp.int32),
        ],
    )(x)
    return out[0].astype(jnp.int64)
